# baseline (speedup 1.0000x reference)
import functools

import jax
import jax.numpy as jnp
from jax import lax
from jax.experimental import pallas as pl
from jax.experimental.pallas import tpu as pltpu
from jax.experimental.pallas import tpu_sc as plsc

N = 10000
E = 320000
D = 128
NPAD = 10240
NC, NS = 2, 16
NW = NC * NS
CHUNK = 128
CH = 80
NPH = 2
PCH = CH // NPH
PHALF = PCH // 2
EPAD = NW * CH * CHUNK
RPT = NPAD // NS

_mesh = plsc.VectorSubcoreMesh(core_axis_name="c", subcore_axis_name="s")


@functools.partial(
    pl.kernel,
    out_type=jax.ShapeDtypeStruct((NC, NPAD, 16), jnp.float32),
    mesh=_mesh,
    scratch_types=[
        pltpu.VMEM((CH, CHUNK), jnp.int32),
        pltpu.VMEM((CHUNK, 16), jnp.float32),
        pltpu.VMEM_SHARED((NPAD, 16), jnp.float32),
    ],
)
def _deg_kernel(dst_hbm, zeros_hbm, out_hbm, idx_v, ones_v, acc_sh):
    cid = lax.axis_index("c")
    sid = lax.axis_index("s")
    wid = cid * NS + sid
    stripe = pl.ds(sid * RPT, RPT)

    def _fill(i, carry):
        ones_v[i] = jnp.ones((16,), jnp.float32)
        return carry

    lax.fori_loop(0, CHUNK, _fill, 0)

    pltpu.sync_copy(zeros_hbm.at[stripe], acc_sh.at[stripe])
    pltpu.sync_copy(dst_hbm.at[wid], idx_v)
    plsc.subcore_barrier()

    def _body(c, carry):
        pltpu.sync_copy(ones_v, acc_sh.at[idx_v.at[c]], add=True)
        return carry

    lax.fori_loop(0, CH, _body, 0)
    plsc.subcore_barrier()
    pltpu.sync_copy(acc_sh.at[stripe], out_hbm.at[cid, stripe])


@functools.partial(
    pl.kernel,
    out_type=jax.ShapeDtypeStruct((NC, NPAD, D), jnp.float32),
    mesh=_mesh,
    scratch_types=[
        pltpu.VMEM((PCH, CHUNK), jnp.int32),
        pltpu.VMEM((PCH, CHUNK), jnp.int32),
        pltpu.VMEM((CHUNK, D), jnp.float32),
        pltpu.VMEM((CHUNK, D), jnp.float32),
        pltpu.VMEM_SHARED((NPAD, D), jnp.float32),
        pltpu.SemaphoreType.DMA,
        pltpu.SemaphoreType.DMA,
    ],
)
def _hop_kernel(t_hbm, src_a, dst_a, src_b, dst_b, out_hbm, src_v, dst_v,
                buf0, buf1, acc_sh, sem0, sem1):
    cid = lax.axis_index("c")
    sid = lax.axis_index("s")
    wid = cid * NS + sid
    stripe = pl.ds(sid * RPT, RPT)

    pltpu.sync_copy(t_hbm.at[stripe], acc_sh.at[stripe])
    plsc.subcore_barrier()

    for src_h, dst_h in ((src_a, dst_a), (src_b, dst_b)):
        pltpu.sync_copy(src_h.at[wid], src_v)
        pltpu.sync_copy(dst_h.at[wid], dst_v)
        pltpu.async_copy(t_hbm.at[src_v.at[0]], buf0, sem0)

        def _body(i, carry):
            c0 = 2 * i
            c1 = c0 + 1
            pltpu.make_async_copy(t_hbm.at[src_v.at[c0]], buf0, sem0).wait()
            pltpu.sync_copy(buf0, acc_sh.at[dst_v.at[c0]], add=True)
            pltpu.async_copy(t_hbm.at[src_v.at[c1]], buf1, sem1).wait()
            pltpu.sync_copy(buf1, acc_sh.at[dst_v.at[c1]], add=True)

            @pl.when(i < PHALF - 1)
            def _():
                pltpu.async_copy(t_hbm.at[src_v.at[c0 + 2]], buf0, sem0)
            return carry

        lax.fori_loop(0, PHALF, _body, 0)
    plsc.subcore_barrier()
    pltpu.sync_copy(acc_sh.at[stripe], out_hbm.at[cid, stripe])


BR = 512


def _deg_block(degp_ref):
    d = degp_ref[0, :, 0:1] + degp_ref[1, :, 0:1] + 1.0
    return d


def _prep_body(degp_ref, x_ref, t0_ref):
    t0_ref[...] = x_ref[...] * lax.rsqrt(_deg_block(degp_ref))


def _mid_body(degp_ref, p_ref, t0_ref, t1_ref):
    s0 = p_ref[0] + p_ref[1] - t0_ref[...]
    t1_ref[...] = s0 / _deg_block(degp_ref)


def _final_body(degp_ref, q_ref, t1_ref, w_ref, b_ref, out_ref):
    s1 = q_ref[0] + q_ref[1] - t1_ref[...]
    h2 = s1 * lax.rsqrt(_deg_block(degp_ref))
    z = lax.dot_general(h2, w_ref[...], (((1,), (1,)), ((), ())),
                        preferred_element_type=jnp.float32) + b_ref[...]
    m = jnp.max(z, axis=1, keepdims=True)
    lse = m + jnp.log(jnp.sum(jnp.exp(z - m), axis=1, keepdims=True))
    out_ref[...] = z - lse


_degp_spec = pl.BlockSpec((NC, BR, 16), lambda i: (0, i, 0))
_row_spec = pl.BlockSpec((BR, D), lambda i: (i, 0))
_pair_spec = pl.BlockSpec((NC, BR, D), lambda i: (0, i, 0))
_grid = (NPAD // BR,)

_prep = pl.pallas_call(
    _prep_body, grid=_grid,
    in_specs=[_degp_spec, _row_spec], out_specs=_row_spec,
    out_shape=jax.ShapeDtypeStruct((NPAD, D), jnp.float32))

_mid = pl.pallas_call(
    _mid_body, grid=_grid,
    in_specs=[_degp_spec, _pair_spec, _row_spec], out_specs=_row_spec,
    out_shape=jax.ShapeDtypeStruct((NPAD, D), jnp.float32))

_final = pl.pallas_call(
    _final_body, grid=_grid,
    in_specs=[_degp_spec, _pair_spec, _row_spec,
              pl.BlockSpec((D, D), lambda i: (0, 0)),
              pl.BlockSpec((1, D), lambda i: (0, 0))],
    out_specs=_row_spec,
    out_shape=jax.ShapeDtypeStruct((NPAD, D), jnp.float32))


def kernel(x, edge_index, W, b):
    src = edge_index[0]
    dst = edge_index[1]
    pad = jnp.full((EPAD - E,), N, dtype=jnp.int32)
    srcs = jnp.concatenate([src, pad]).reshape(NW, NPH, PCH, CHUNK)
    dsts = jnp.concatenate([dst, pad]).reshape(NW, NPH, PCH, CHUNK)
    src_a, src_b = srcs[:, 0], srcs[:, 1]
    dst_a, dst_b = dsts[:, 0], dsts[:, 1]
    x_pad = jnp.zeros((NPAD, D), jnp.float32).at[:N].set(x)
    zeros16 = jnp.zeros((NPAD, 16), jnp.float32)

    degp = _deg_kernel(dsts.reshape(NW, CH, CHUNK), zeros16)
    t0 = _prep(degp, x_pad)
    p = _hop_kernel(t0, src_a, dst_a, src_b, dst_b)
    t1 = _mid(degp, p, t0)
    q = _hop_kernel(t1, src_a, dst_a, src_b, dst_b)
    out = _final(degp, q, t1, W, b.reshape(1, D))
    return out[:N]


# speedup vs baseline: 1.0578x; 1.0578x over previous
import functools

import jax
import jax.numpy as jnp
from jax import lax
from jax.experimental import pallas as pl
from jax.experimental.pallas import tpu as pltpu
from jax.experimental.pallas import tpu_sc as plsc

N = 10000
E = 320000
D = 128
NPAD = 10240
NC, NS = 2, 16
NW = NC * NS
CHUNK = 128
CH = 80
DCHUNK = 128
DCH = 80
EPAD = NW * CH * CHUNK
RPT = NPAD // NS

_mesh = plsc.VectorSubcoreMesh(core_axis_name="c", subcore_axis_name="s")


@functools.partial(
    pl.kernel,
    out_type=jax.ShapeDtypeStruct((NC, NPAD, 16), jnp.float32),
    mesh=_mesh,
    scratch_types=[
        pltpu.VMEM((DCH, DCHUNK), jnp.int32),
        pltpu.VMEM((DCHUNK, 16), jnp.float32),
        pltpu.VMEM_SHARED((NPAD, 16), jnp.float32),
    ],
)
def _deg_kernel(dst_hbm, zeros_hbm, out_hbm, idx_v, ones_v, acc_sh):
    cid = lax.axis_index("c")
    sid = lax.axis_index("s")
    wid = cid * NS + sid
    stripe = pl.ds(sid * RPT, RPT)

    def _fill(i, carry):
        ones_v[i] = jnp.ones((16,), jnp.float32)
        return carry

    lax.fori_loop(0, DCHUNK, _fill, 0)

    pltpu.sync_copy(zeros_hbm.at[stripe], acc_sh.at[stripe])
    pltpu.sync_copy(dst_hbm.at[wid], idx_v)
    plsc.subcore_barrier()

    def _body(c, carry):
        pltpu.sync_copy(ones_v, acc_sh.at[idx_v.at[c]], add=True)
        return carry

    lax.fori_loop(0, DCH, _body, 0)
    plsc.subcore_barrier()
    pltpu.sync_copy(acc_sh.at[stripe], out_hbm.at[cid, stripe])


@functools.partial(
    pl.kernel,
    out_type=jax.ShapeDtypeStruct((NC, NPAD, D), jnp.float32),
    mesh=_mesh,
    scratch_types=[
        pltpu.VMEM((CH, CHUNK), jnp.int32),
        pltpu.VMEM((CH, CHUNK), jnp.int32),
        pltpu.VMEM((CHUNK, D), jnp.float32),
        pltpu.VMEM_SHARED((NPAD, D), jnp.float32),
        pltpu.SemaphoreType.DMA,
    ],
)
def _hop_kernel(t_hbm, src_hbm, dst_hbm, out_hbm, src_v, dst_v,
                buf0, acc_sh, sem0):
    cid = lax.axis_index("c")
    sid = lax.axis_index("s")
    wid = cid * NS + sid
    stripe = pl.ds(sid * RPT, RPT)

    pltpu.sync_copy(t_hbm.at[stripe], acc_sh.at[stripe])
    plsc.subcore_barrier()

    pltpu.sync_copy(src_hbm.at[wid], src_v)
    pltpu.sync_copy(dst_hbm.at[wid], dst_v)

    def _body(c, carry):
        pltpu.async_copy(t_hbm.at[src_v.at[c]], buf0, sem0).wait()
        pltpu.sync_copy(buf0, acc_sh.at[dst_v.at[c]], add=True)
        return carry

    lax.fori_loop(0, CH, _body, 0)
    plsc.subcore_barrier()
    pltpu.sync_copy(acc_sh.at[stripe], out_hbm.at[cid, stripe])


BR = 512


def _deg_block(degp_ref):
    d = degp_ref[0, :, 0:1] + degp_ref[1, :, 0:1] + 1.0
    return d


def _prep_body(degp_ref, x_ref, t0_ref):
    t0_ref[...] = x_ref[...] * lax.rsqrt(_deg_block(degp_ref))


def _mid_body(degp_ref, p_ref, t0_ref, t1_ref):
    s0 = p_ref[0] + p_ref[1] - t0_ref[...]
    t1_ref[...] = s0 / _deg_block(degp_ref)


def _final_body(degp_ref, q_ref, t1_ref, w_ref, b_ref, out_ref):
    s1 = q_ref[0] + q_ref[1] - t1_ref[...]
    h2 = s1 * lax.rsqrt(_deg_block(degp_ref))
    z = lax.dot_general(h2, w_ref[...], (((1,), (1,)), ((), ())),
                        preferred_element_type=jnp.float32) + b_ref[...]
    m = jnp.max(z, axis=1, keepdims=True)
    lse = m + jnp.log(jnp.sum(jnp.exp(z - m), axis=1, keepdims=True))
    out_ref[...] = z - lse


_degp_spec = pl.BlockSpec((NC, BR, 16), lambda i: (0, i, 0))
_row_spec = pl.BlockSpec((BR, D), lambda i: (i, 0))
_pair_spec = pl.BlockSpec((NC, BR, D), lambda i: (0, i, 0))
_grid = (NPAD // BR,)

_prep = pl.pallas_call(
    _prep_body, grid=_grid,
    in_specs=[_degp_spec, _row_spec], out_specs=_row_spec,
    out_shape=jax.ShapeDtypeStruct((NPAD, D), jnp.float32))

_mid = pl.pallas_call(
    _mid_body, grid=_grid,
    in_specs=[_degp_spec, _pair_spec, _row_spec], out_specs=_row_spec,
    out_shape=jax.ShapeDtypeStruct((NPAD, D), jnp.float32))

_final = pl.pallas_call(
    _final_body, grid=_grid,
    in_specs=[_degp_spec, _pair_spec, _row_spec,
              pl.BlockSpec((D, D), lambda i: (0, 0)),
              pl.BlockSpec((1, D), lambda i: (0, 0))],
    out_specs=_row_spec,
    out_shape=jax.ShapeDtypeStruct((NPAD, D), jnp.float32))


def kernel(x, edge_index, W, b):
    src = edge_index[0]
    dst = edge_index[1]
    pad = jnp.full((EPAD - E,), N, dtype=jnp.int32)
    srcs = jnp.concatenate([src, pad]).reshape(NW, CH, CHUNK)
    dsts = jnp.concatenate([dst, pad]).reshape(NW, CH, CHUNK)
    x_pad = jnp.zeros((NPAD, D), jnp.float32).at[:N].set(x)
    zeros16 = jnp.zeros((NPAD, 16), jnp.float32)

    degp = _deg_kernel(dsts.reshape(NW, DCH, DCHUNK), zeros16)
    t0 = _prep(degp, x_pad)
    p = _hop_kernel(t0, srcs, dsts)
    t1 = _mid(degp, p, t0)
    q = _hop_kernel(t1, srcs, dsts)
    out = _final(degp, q, t1, W, b.reshape(1, D))
    return out[:N]


# speedup vs baseline: 2.5017x; 2.3650x over previous
import functools

import jax
import jax.numpy as jnp
from jax import lax
from jax.experimental import pallas as pl
from jax.experimental.pallas import tpu as pltpu
from jax.experimental.pallas import tpu_sc as plsc

N = 10000
E = 320000
D = 128
NPAD = 10240
NC, NS = 2, 16
NW = NC * NS
CHUNK = 128
CH = 80
DCHUNK = 128
DCH = 80
EPAD = NW * CH * CHUNK
RPT = NPAD // NS

_mesh = plsc.VectorSubcoreMesh(core_axis_name="c", subcore_axis_name="s")


@functools.partial(
    pl.kernel,
    out_type=jax.ShapeDtypeStruct((NC, NPAD, 16), jnp.float32),
    mesh=_mesh,
    scratch_types=[
        pltpu.VMEM((DCH, DCHUNK), jnp.int32),
        pltpu.VMEM((DCHUNK, 16), jnp.float32),
        pltpu.VMEM_SHARED((NPAD, 16), jnp.float32),
    ],
)
def _deg_kernel(dst_hbm, zeros_hbm, out_hbm, idx_v, ones_v, acc_sh):
    cid = lax.axis_index("c")
    sid = lax.axis_index("s")
    wid = cid * NS + sid
    stripe = pl.ds(sid * RPT, RPT)

    def _fill(i, carry):
        ones_v[i] = jnp.ones((16,), jnp.float32)
        return carry

    lax.fori_loop(0, DCHUNK, _fill, 0)

    pltpu.sync_copy(zeros_hbm.at[stripe], acc_sh.at[stripe])
    pltpu.sync_copy(dst_hbm.at[wid], idx_v)
    plsc.subcore_barrier()

    def _body(c, carry):
        pltpu.sync_copy(ones_v, acc_sh.at[idx_v.at[c]], add=True)
        return carry

    lax.fori_loop(0, DCH, _body, 0)
    plsc.subcore_barrier()
    pltpu.sync_copy(acc_sh.at[stripe], out_hbm.at[cid, stripe])


@functools.partial(
    pl.kernel,
    out_type=jax.ShapeDtypeStruct((NC, NPAD, D), jnp.float32),
    mesh=_mesh,
    scratch_types=[
        pltpu.VMEM((CH, CHUNK), jnp.int32),
        pltpu.VMEM((CH, CHUNK), jnp.int32),
        pltpu.VMEM((CHUNK, D), jnp.float32),
        pltpu.VMEM_SHARED((NPAD, D), jnp.float32),
        pltpu.SemaphoreType.DMA,
    ],
)
def _hop_kernel(t_hbm, src_hbm, dst_hbm, out_hbm, src_v, dst_v,
                buf0, acc_sh, sem0):
    cid = lax.axis_index("c")
    sid = lax.axis_index("s")
    wid = cid * NS + sid
    stripe = pl.ds(sid * RPT, RPT)

    pltpu.sync_copy(t_hbm.at[stripe], acc_sh.at[stripe])
    plsc.subcore_barrier()

    pltpu.sync_copy(src_hbm.at[wid], src_v)
    pltpu.sync_copy(dst_hbm.at[wid], dst_v)

    def _body(c, carry):
        pltpu.async_copy(t_hbm.at[src_v.at[c]], buf0, sem0).wait()
        pltpu.sync_copy(buf0, acc_sh.at[dst_v.at[c]], add=True)
        return carry

    lax.fori_loop(0, CH, _body, 0)
    plsc.subcore_barrier()
    pltpu.sync_copy(acc_sh.at[stripe], out_hbm.at[cid, stripe])


BR = 512


def _deg_block(degp_ref):
    d = degp_ref[0, :, 0:1] + degp_ref[1, :, 0:1] + 1.0
    return d


def _prep_body(degp_ref, x_ref, t0_ref):
    t0_ref[...] = x_ref[...] * lax.rsqrt(_deg_block(degp_ref))


def _mid_body(degp_ref, p_ref, t0_ref, t1_ref):
    s0 = p_ref[0] + p_ref[1] - t0_ref[...]
    t1_ref[...] = s0 / _deg_block(degp_ref)


def _final_body(degp_ref, q_ref, t1_ref, w_ref, b_ref, out_ref):
    s1 = q_ref[0] + q_ref[1] - t1_ref[...]
    h2 = s1 * lax.rsqrt(_deg_block(degp_ref))
    z = lax.dot_general(h2, w_ref[...], (((1,), (1,)), ((), ())),
                        preferred_element_type=jnp.float32) + b_ref[...]
    m = jnp.max(z, axis=1, keepdims=True)
    lse = m + jnp.log(jnp.sum(jnp.exp(z - m), axis=1, keepdims=True))
    out_ref[...] = z - lse


_degp_spec = pl.BlockSpec((NC, BR, 16), lambda i: (0, i, 0))
_row_spec = pl.BlockSpec((BR, D), lambda i: (i, 0))
_pair_spec = pl.BlockSpec((NC, BR, D), lambda i: (0, i, 0))
_grid = (NPAD // BR,)

_prep = pl.pallas_call(
    _prep_body, grid=_grid,
    in_specs=[_degp_spec, _row_spec], out_specs=_row_spec,
    out_shape=jax.ShapeDtypeStruct((NPAD, D), jnp.float32))

_mid = pl.pallas_call(
    _mid_body, grid=_grid,
    in_specs=[_degp_spec, _pair_spec, _row_spec], out_specs=_row_spec,
    out_shape=jax.ShapeDtypeStruct((NPAD, D), jnp.float32))

_final = pl.pallas_call(
    _final_body, grid=_grid,
    in_specs=[_degp_spec, _pair_spec, _row_spec,
              pl.BlockSpec((D, D), lambda i: (0, 0)),
              pl.BlockSpec((1, D), lambda i: (0, 0))],
    out_specs=_row_spec,
    out_shape=jax.ShapeDtypeStruct((NPAD, D), jnp.float32))


def kernel(x, edge_index, W, b):
    src = edge_index[0]
    dst = edge_index[1]
    pad = N + (jnp.arange(EPAD - E, dtype=jnp.int32) % (NPAD - N))
    srcs = jnp.concatenate([src, pad]).reshape(NW, CH, CHUNK)
    dsts = jnp.concatenate([dst, pad]).reshape(NW, CH, CHUNK)
    x_pad = jnp.zeros((NPAD, D), jnp.float32).at[:N].set(x)
    zeros16 = jnp.zeros((NPAD, 16), jnp.float32)

    degp = _deg_kernel(dsts.reshape(NW, DCH, DCHUNK), zeros16)
    t0 = _prep(degp, x_pad)
    p = _hop_kernel(t0, srcs, dsts)
    t1 = _mid(degp, p, t0)
    q = _hop_kernel(t1, srcs, dsts)
    out = _final(degp, q, t1, W, b.reshape(1, D))
    return out[:N]


# speedup vs baseline: 3.4678x; 1.3862x over previous
import functools

import jax
import jax.numpy as jnp
from jax import lax
from jax.experimental import pallas as pl
from jax.experimental.pallas import tpu as pltpu
from jax.experimental.pallas import tpu_sc as plsc

N = 10000
E = 320000
D = 128
NPAD = 10240
NC, NS = 2, 16
NW = NC * NS
CHUNK = 128
CH = 80
NPH = 2
PCH = CH // NPH
PHALF = PCH // 2
DCHUNK = 128
DCH = 80
EPAD = NW * CH * CHUNK
RPT = NPAD // NS

_mesh = plsc.VectorSubcoreMesh(core_axis_name="c", subcore_axis_name="s")


@functools.partial(
    pl.kernel,
    out_type=jax.ShapeDtypeStruct((NC, NPAD, 16), jnp.float32),
    mesh=_mesh,
    scratch_types=[
        pltpu.VMEM((DCH, DCHUNK), jnp.int32),
        pltpu.VMEM((DCHUNK, 16), jnp.float32),
        pltpu.VMEM_SHARED((NPAD, 16), jnp.float32),
    ],
)
def _deg_kernel(dst_hbm, zeros_hbm, out_hbm, idx_v, ones_v, acc_sh):
    cid = lax.axis_index("c")
    sid = lax.axis_index("s")
    wid = cid * NS + sid
    stripe = pl.ds(sid * RPT, RPT)

    def _fill(i, carry):
        ones_v[i] = jnp.ones((16,), jnp.float32)
        return carry

    lax.fori_loop(0, DCHUNK, _fill, 0)

    pltpu.sync_copy(zeros_hbm.at[stripe], acc_sh.at[stripe])
    pltpu.sync_copy(dst_hbm.at[wid], idx_v)
    plsc.subcore_barrier()

    def _body(c, carry):
        pltpu.sync_copy(ones_v, acc_sh.at[idx_v.at[c]], add=True)
        return carry

    lax.fori_loop(0, DCH, _body, 0)
    plsc.subcore_barrier()
    pltpu.sync_copy(acc_sh.at[stripe], out_hbm.at[cid, stripe])


@functools.partial(
    pl.kernel,
    out_type=jax.ShapeDtypeStruct((NC, NPAD, D), jnp.float32),
    mesh=_mesh,
    scratch_types=[
        pltpu.VMEM((PCH, CHUNK), jnp.int32),
        pltpu.VMEM((PCH, CHUNK), jnp.int32),
        pltpu.VMEM((CHUNK, D), jnp.float32),
        pltpu.VMEM((CHUNK, D), jnp.float32),
        pltpu.VMEM_SHARED((NPAD, D), jnp.float32),
        pltpu.SemaphoreType.DMA,
        pltpu.SemaphoreType.DMA,
    ],
)
def _hop_kernel(t_hbm, src_a, dst_a, src_b, dst_b, out_hbm, src_v, dst_v,
                buf0, buf1, acc_sh, sem0, sem1):
    cid = lax.axis_index("c")
    sid = lax.axis_index("s")
    wid = cid * NS + sid
    stripe = pl.ds(sid * RPT, RPT)

    pltpu.sync_copy(t_hbm.at[stripe], acc_sh.at[stripe])
    plsc.subcore_barrier()

    for src_h, dst_h in ((src_a, dst_a), (src_b, dst_b)):
        pltpu.sync_copy(src_h.at[wid], src_v)
        pltpu.sync_copy(dst_h.at[wid], dst_v)
        pltpu.async_copy(t_hbm.at[src_v.at[0]], buf0, sem0)

        def _body(i, carry):
            c0 = 2 * i
            c1 = c0 + 1
            pltpu.async_copy(t_hbm.at[src_v.at[c1]], buf1, sem1)
            pltpu.make_async_copy(t_hbm.at[src_v.at[c0]], buf0, sem0).wait()
            pltpu.sync_copy(buf0, acc_sh.at[dst_v.at[c0]], add=True)

            @pl.when(i < PHALF - 1)
            def _():
                pltpu.async_copy(t_hbm.at[src_v.at[c0 + 2]], buf0, sem0)

            pltpu.make_async_copy(t_hbm.at[src_v.at[c1]], buf1, sem1).wait()
            pltpu.sync_copy(buf1, acc_sh.at[dst_v.at[c1]], add=True)
            return carry

        lax.fori_loop(0, PHALF, _body, 0)
    plsc.subcore_barrier()
    pltpu.sync_copy(acc_sh.at[stripe], out_hbm.at[cid, stripe])


BR = 512


def _deg_block(degp_ref):
    d = degp_ref[0, :, 0:1] + degp_ref[1, :, 0:1] + 1.0
    return d


def _prep_body(degp_ref, x_ref, t0_ref):
    t0_ref[...] = x_ref[...] * lax.rsqrt(_deg_block(degp_ref))


def _mid_body(degp_ref, p_ref, t0_ref, t1_ref):
    s0 = p_ref[0] + p_ref[1] - t0_ref[...]
    t1_ref[...] = s0 / _deg_block(degp_ref)


def _final_body(degp_ref, q_ref, t1_ref, w_ref, b_ref, out_ref):
    s1 = q_ref[0] + q_ref[1] - t1_ref[...]
    h2 = s1 * lax.rsqrt(_deg_block(degp_ref))
    z = lax.dot_general(h2, w_ref[...], (((1,), (1,)), ((), ())),
                        preferred_element_type=jnp.float32) + b_ref[...]
    m = jnp.max(z, axis=1, keepdims=True)
    lse = m + jnp.log(jnp.sum(jnp.exp(z - m), axis=1, keepdims=True))
    out_ref[...] = z - lse


_degp_spec = pl.BlockSpec((NC, BR, 16), lambda i: (0, i, 0))
_row_spec = pl.BlockSpec((BR, D), lambda i: (i, 0))
_pair_spec = pl.BlockSpec((NC, BR, D), lambda i: (0, i, 0))
_grid = (NPAD // BR,)

_prep = pl.pallas_call(
    _prep_body, grid=_grid,
    in_specs=[_degp_spec, _row_spec], out_specs=_row_spec,
    out_shape=jax.ShapeDtypeStruct((NPAD, D), jnp.float32))

_mid = pl.pallas_call(
    _mid_body, grid=_grid,
    in_specs=[_degp_spec, _pair_spec, _row_spec], out_specs=_row_spec,
    out_shape=jax.ShapeDtypeStruct((NPAD, D), jnp.float32))

_final = pl.pallas_call(
    _final_body, grid=_grid,
    in_specs=[_degp_spec, _pair_spec, _row_spec,
              pl.BlockSpec((D, D), lambda i: (0, 0)),
              pl.BlockSpec((1, D), lambda i: (0, 0))],
    out_specs=_row_spec,
    out_shape=jax.ShapeDtypeStruct((NPAD, D), jnp.float32))


def kernel(x, edge_index, W, b):
    src = edge_index[0]
    dst = edge_index[1]
    pad = N + (jnp.arange(EPAD - E, dtype=jnp.int32) % (NPAD - N))
    srcs = jnp.concatenate([src, pad]).reshape(NW, NPH, PCH, CHUNK)
    dsts = jnp.concatenate([dst, pad]).reshape(NW, NPH, PCH, CHUNK)
    src_a, src_b = srcs[:, 0], srcs[:, 1]
    dst_a, dst_b = dsts[:, 0], dsts[:, 1]
    x_pad = jnp.zeros((NPAD, D), jnp.float32).at[:N].set(x)
    zeros16 = jnp.zeros((NPAD, 16), jnp.float32)

    degp = _deg_kernel(dsts.reshape(NW, DCH, DCHUNK), zeros16)
    t0 = _prep(degp, x_pad)
    p = _hop_kernel(t0, src_a, dst_a, src_b, dst_b)
    t1 = _mid(degp, p, t0)
    q = _hop_kernel(t1, src_a, dst_a, src_b, dst_b)
    out = _final(degp, q, t1, W, b.reshape(1, D))
    return out[:N]


# speedup vs baseline: 3.4837x; 1.0046x over previous
import functools

import jax
import jax.numpy as jnp
from jax import lax
from jax.experimental import pallas as pl
from jax.experimental.pallas import tpu as pltpu
from jax.experimental.pallas import tpu_sc as plsc

N = 10000
E = 320000
D = 128
NPAD = 10240
NC, NS = 2, 16
NW = NC * NS
CHUNK = 128
CH = 80
NPH = 2
PCH = CH // NPH
PHALF = PCH // 2
DCHUNK = 128
DCH = 80
EPAD = NW * CH * CHUNK
RPT = NPAD // NS

_mesh = plsc.VectorSubcoreMesh(core_axis_name="c", subcore_axis_name="s")


@functools.partial(
    pl.kernel,
    out_type=jax.ShapeDtypeStruct((NC, NPAD, 16), jnp.float32),
    mesh=_mesh,
    scratch_types=[
        pltpu.VMEM((DCH, DCHUNK), jnp.int32),
        pltpu.VMEM((DCHUNK, 16), jnp.float32),
        pltpu.VMEM_SHARED((NPAD, 16), jnp.float32),
        pltpu.SemaphoreType.DMA,
    ],
)
def _deg_kernel(dst_hbm, zeros_hbm, out_hbm, idx_v, ones_v, acc_sh, dsem):
    cid = lax.axis_index("c")
    sid = lax.axis_index("s")
    wid = cid * NS + sid
    stripe = pl.ds(sid * RPT, RPT)

    def _fill(i, carry):
        ones_v[i] = jnp.ones((16,), jnp.float32)
        return carry

    lax.fori_loop(0, DCHUNK, _fill, 0)

    pltpu.sync_copy(zeros_hbm.at[stripe], acc_sh.at[stripe])
    pltpu.sync_copy(dst_hbm.at[wid], idx_v)
    plsc.subcore_barrier()

    def _body(c, carry):
        pltpu.sync_copy(ones_v, acc_sh.at[idx_v.at[c]], add=True)
        return carry

    lax.fori_loop(0, DCH, _body, 0)
    plsc.subcore_barrier()
    pltpu.sync_copy(acc_sh.at[stripe], out_hbm.at[cid, stripe])


@functools.partial(
    pl.kernel,
    out_type=jax.ShapeDtypeStruct((NC, NPAD, D), jnp.float32),
    mesh=_mesh,
    scratch_types=[
        pltpu.VMEM((PCH, CHUNK), jnp.int32),
        pltpu.VMEM((PCH, CHUNK), jnp.int32),
        pltpu.VMEM((CHUNK, D), jnp.float32),
        pltpu.VMEM((CHUNK, D), jnp.float32),
        pltpu.VMEM_SHARED((NPAD, D), jnp.float32),
        pltpu.SemaphoreType.DMA,
        pltpu.SemaphoreType.DMA,
        pltpu.SemaphoreType.DMA,
        pltpu.SemaphoreType.DMA,
    ],
)
def _hop_kernel(t_hbm, src_a, dst_a, src_b, dst_b, out_hbm, src_v, dst_v,
                buf0, buf1, acc_sh, sem0, sem1, ssem0, ssem1):
    cid = lax.axis_index("c")
    sid = lax.axis_index("s")
    wid = cid * NS + sid
    stripe = pl.ds(sid * RPT, RPT)

    pltpu.sync_copy(t_hbm.at[stripe], acc_sh.at[stripe])
    plsc.subcore_barrier()

    for src_h, dst_h in ((src_a, dst_a), (src_b, dst_b)):
        pltpu.sync_copy(src_h.at[wid], src_v)
        pltpu.sync_copy(dst_h.at[wid], dst_v)
        pltpu.async_copy(t_hbm.at[src_v.at[0]], buf0, sem0)
        pltpu.async_copy(t_hbm.at[src_v.at[1]], buf1, sem1)

        def _body(i, carry):
            c0 = 2 * i
            c1 = c0 + 1
            pltpu.make_async_copy(t_hbm.at[src_v.at[c0]], buf0, sem0).wait()
            pltpu.sync_copy(buf0, acc_sh.at[dst_v.at[c0]], add=True)

            @pl.when(i < PHALF - 1)
            def _():
                pltpu.async_copy(t_hbm.at[src_v.at[c0 + 2]], buf0, sem0)

            pltpu.make_async_copy(t_hbm.at[src_v.at[c1]], buf1, sem1).wait()
            pltpu.sync_copy(buf1, acc_sh.at[dst_v.at[c1]], add=True)

            @pl.when(i < PHALF - 1)
            def _():
                pltpu.async_copy(t_hbm.at[src_v.at[c0 + 3]], buf1, sem1)
            return carry

        lax.fori_loop(0, PHALF, _body, 0)
    plsc.subcore_barrier()
    pltpu.sync_copy(acc_sh.at[stripe], out_hbm.at[cid, stripe])


BR = 512


def _deg_block(degp_ref):
    d = degp_ref[0, :, 0:1] + degp_ref[1, :, 0:1] + 1.0
    return d


def _prep_body(degp_ref, x_ref, t0_ref):
    t0_ref[...] = x_ref[...] * lax.rsqrt(_deg_block(degp_ref))


def _mid_body(degp_ref, p_ref, t0_ref, t1_ref):
    s0 = p_ref[0] + p_ref[1] - t0_ref[...]
    t1_ref[...] = s0 / _deg_block(degp_ref)


def _final_body(degp_ref, q_ref, t1_ref, w_ref, b_ref, out_ref):
    s1 = q_ref[0] + q_ref[1] - t1_ref[...]
    h2 = s1 * lax.rsqrt(_deg_block(degp_ref))
    z = lax.dot_general(h2, w_ref[...], (((1,), (1,)), ((), ())),
                        preferred_element_type=jnp.float32) + b_ref[...]
    m = jnp.max(z, axis=1, keepdims=True)
    lse = m + jnp.log(jnp.sum(jnp.exp(z - m), axis=1, keepdims=True))
    out_ref[...] = z - lse


_degp_spec = pl.BlockSpec((NC, BR, 16), lambda i: (0, i, 0))
_row_spec = pl.BlockSpec((BR, D), lambda i: (i, 0))
_pair_spec = pl.BlockSpec((NC, BR, D), lambda i: (0, i, 0))
_grid = (NPAD // BR,)

_prep = pl.pallas_call(
    _prep_body, grid=_grid,
    in_specs=[_degp_spec, _row_spec], out_specs=_row_spec,
    out_shape=jax.ShapeDtypeStruct((NPAD, D), jnp.float32))

_mid = pl.pallas_call(
    _mid_body, grid=_grid,
    in_specs=[_degp_spec, _pair_spec, _row_spec], out_specs=_row_spec,
    out_shape=jax.ShapeDtypeStruct((NPAD, D), jnp.float32))

_final = pl.pallas_call(
    _final_body, grid=_grid,
    in_specs=[_degp_spec, _pair_spec, _row_spec,
              pl.BlockSpec((D, D), lambda i: (0, 0)),
              pl.BlockSpec((1, D), lambda i: (0, 0))],
    out_specs=_row_spec,
    out_shape=jax.ShapeDtypeStruct((NPAD, D), jnp.float32))


def kernel(x, edge_index, W, b):
    src = edge_index[0]
    dst = edge_index[1]
    pad = N + (jnp.arange(EPAD - E, dtype=jnp.int32) % (NPAD - N))
    srcs = jnp.concatenate([src, pad]).reshape(NW, NPH, PCH, CHUNK)
    dsts = jnp.concatenate([dst, pad]).reshape(NW, NPH, PCH, CHUNK)
    src_a, src_b = srcs[:, 0], srcs[:, 1]
    dst_a, dst_b = dsts[:, 0], dsts[:, 1]
    x_pad = jnp.zeros((NPAD, D), jnp.float32).at[:N].set(x)
    zeros16 = jnp.zeros((NPAD, 16), jnp.float32)

    degp = _deg_kernel(dsts.reshape(NW, DCH, DCHUNK), zeros16)
    t0 = _prep(degp, x_pad)
    p = _hop_kernel(t0, src_a, dst_a, src_b, dst_b)
    t1 = _mid(degp, p, t0)
    q = _hop_kernel(t1, src_a, dst_a, src_b, dst_b)
    out = _final(degp, q, t1, W, b.reshape(1, D))
    return out[:N]
